# baseline (speedup 1.0000x reference)
import functools

import jax
import jax.numpy as jnp
from jax import lax
from jax.experimental import pallas as pl
from jax.experimental.pallas import tpu as pltpu
from jax.experimental.pallas import tpu_sc as plsc

_S, _B, _D = 2048, 4, 2048
_NW = 32
_S_PER_W = _S // _NW
_CH = 2
_NBUF = 6
_NCH = _S_PER_W // _CH
_PF = 4
_L = 16


def _sc_body(x_hbm, w_hbm, o_hbm, *refs):
    xbufs = refs[0:_NBUF]
    wbufs = refs[_NBUF:2 * _NBUF]
    sin = refs[2 * _NBUF:3 * _NBUF]
    sout = refs[3 * _NBUF:4 * _NBUF]

    wid = lax.axis_index("s") * 2 + lax.axis_index("c")
    base0 = wid * _S_PER_W

    def start_in(c, p):
        base = base0 + c * _CH
        pltpu.make_async_copy(x_hbm.at[pl.ds(base, _CH)], xbufs[p], sin[p]).start()
        pltpu.make_async_copy(w_hbm.at[pl.ds(base, _CH)], wbufs[p], sin[p]).start()

    def wait_in(c, p):
        base = base0 + c * _CH
        pltpu.make_async_copy(x_hbm.at[pl.ds(base, _CH)], xbufs[p], sin[p]).wait()
        pltpu.make_async_copy(w_hbm.at[pl.ds(base, _CH)], wbufs[p], sin[p]).wait()

    def start_out(c, p):
        base = base0 + c * _CH
        pltpu.make_async_copy(xbufs[p], o_hbm.at[pl.ds(base, _CH)], sout[p]).start()

    def wait_out(c, p):
        base = base0 + c * _CH
        pltpu.make_async_copy(xbufs[p], o_hbm.at[pl.ds(base, _CH)], sout[p]).wait()

    def compute(p):
        xb, wb = xbufs[p], wbufs[p]

        def col(i, carry):
            off = i * _L
            for s in range(_CH):
                w = wb[s, pl.ds(off, _L)]
                for b in range(_B):
                    plsc.addupdate(xb.at[s, b, pl.ds(off, _L)], w)
            return carry

        lax.fori_loop(0, _D // _L, col, 0, unroll=2)

    for c in range(_PF):
        start_in(c, c % _NBUF)

    for c in range(_NCH):
        nxt = c + _PF
        if nxt < _NCH:
            if nxt - _NBUF >= 0:
                wait_out(nxt - _NBUF, nxt % _NBUF)
            start_in(nxt, nxt % _NBUF)
        wait_in(c, c % _NBUF)
        compute(c % _NBUF)
        start_out(c, c % _NBUF)

    for c in range(_NCH - _NBUF, _NCH):
        wait_out(c, c % _NBUF)


def kernel(x, W):
    mesh = plsc.VectorSubcoreMesh(core_axis_name="c", subcore_axis_name="s")
    k = functools.partial(
        pl.kernel,
        mesh=mesh,
        out_type=jax.ShapeDtypeStruct((_S, _B, _D), jnp.float32),
        scratch_types=(
            [pltpu.VMEM((_CH, _B, _D), jnp.float32) for _ in range(_NBUF)]
            + [pltpu.VMEM((_CH, _D), jnp.float32) for _ in range(_NBUF)]
            + [pltpu.SemaphoreType.DMA for _ in range(2 * _NBUF)]
        ),
    )(_sc_body)
    return k(x, W)
